# baseline (speedup 1.0000x reference)
import functools

import jax
import jax.numpy as jnp
from jax import lax
from jax.experimental import pallas as pl
from jax.experimental.pallas import tpu as pltpu
from jax.experimental.pallas import tpu_sc as plsc

_NEG_SLOPE = 0.2
_L = 16


def _make_sc_kernel(n, cw, e0=0):
    info = plsc.get_sparse_core_info()
    nc, ns = info.num_cores, info.num_subcores
    nw = nc * ns
    ew = n // nw
    ch = 80
    nch = ew // ch
    assert n == nw * ew and ew == ch * nch and cw % _L == 0 and nch % 2 == 1
    nb = 2 * cw // _L

    mesh = plsc.VectorSubcoreMesh(core_axis_name="c", subcore_axis_name="s")
    scratch = (
        [pltpu.VMEM((ch * cw,), jnp.float32)] * 4
        + [
            pltpu.VMEM((2 * cw,), jnp.float32),
            pltpu.VMEM((_L * _L,), jnp.float32),
            pltpu.VMEM((ew,), jnp.float32),
        ]
        + [pltpu.SemaphoreType.DMA] * 4
    )

    @functools.partial(
        pl.kernel,
        mesh=mesh,
        out_type=jax.ShapeDtypeStruct((n,), jnp.float32),
        scratch_types=scratch,
        compiler_params=pltpu.CompilerParams(needs_layout_passes=False),
    )
    def k(wi_hbm, wj_hbm, asrc_hbm, adst_hbm, out_hbm,
          wi0, wi1, wj0, wj1, avec, pbuf, outbuf, si0, si1, sj0, sj1):
        wid = lax.axis_index("s") * nc + lax.axis_index("c")
        base = wid * ew

        pltpu.sync_copy(asrc_hbm, avec.at[pl.ds(0, cw)])
        pltpu.sync_copy(adst_hbm, avec.at[pl.ds(cw, cw)])
        a_regs = [avec[pl.ds(b * _L, _L)] for b in range(nb)]
        tidx = lax.iota(jnp.int32, _L) * _L

        slots = [(wi0, wj0, si0, sj0), (wi1, wj1, si1, sj1)]

        def start(c, slot):
            wi_b, wj_b, si, sj = slots[slot]
            off = (e0 + base + c * ch) * cw
            pltpu.async_copy(wi_hbm.at[pl.ds(off, ch * cw)], wi_b, si)
            pltpu.async_copy(wj_hbm.at[pl.ds(off, ch * cw)], wj_b, sj)

        def wait(c, slot):
            wi_b, wj_b, si, sj = slots[slot]
            off = (e0 + base + c * ch) * cw
            pltpu.make_async_copy(wi_hbm.at[pl.ds(off, ch * cw)], wi_b, si).wait()
            pltpu.make_async_copy(wj_hbm.at[pl.ds(off, ch * cw)], wj_b, sj).wait()

        def compute(c, slot):
            wi_b, wj_b, _, _ = slots[slot]

            def group(g, carry):
                for e in range(_L):
                    ro = (g * _L + e) * cw
                    prods = [wi_b[pl.ds(ro + b * _L, _L)] * a_regs[b]
                             for b in range(nb // 2)]
                    prods += [wj_b[pl.ds(ro + b * _L, _L)] * a_regs[nb // 2 + b]
                              for b in range(nb // 2)]
                    while len(prods) > 1:
                        nxt = [prods[i] + prods[i + 1]
                               for i in range(0, len(prods) - 1, 2)]
                        if len(prods) % 2:
                            nxt.append(prods[-1])
                        prods = nxt
                    pbuf[pl.ds(e * _L, _L)] = prods[0]
                cols = [plsc.load_gather(pbuf, [tidx + l]) for l in range(_L)]
                while len(cols) > 1:
                    cols = [cols[i] + cols[i + 1] for i in range(0, len(cols), 2)]
                r = cols[0]
                r = jnp.where(r >= 0, r, _NEG_SLOPE * r)
                outbuf[pl.ds(c * ch + g * _L, _L)] = r
                return carry

            lax.fori_loop(0, ch // _L, group, 0)

        start(0, 0)

        def body(t, carry):
            c0 = 2 * t
            start(c0 + 1, 1)
            wait(c0, 0)
            compute(c0, 0)
            start(c0 + 2, 0)
            wait(c0 + 1, 1)
            compute(c0 + 1, 1)
            return carry

        lax.fori_loop(0, (nch - 1) // 2, body, 0)
        wait(nch - 1, 0)
        compute(nch - 1, 0)
        pltpu.sync_copy(outbuf, out_hbm.at[pl.ds(base, ew)])

    return k


def _make_tc_kernel(n_tc, cw, blk):
    grid = pl.cdiv(n_tc, blk)

    def body(wi_ref, wj_ref, as_ref, ad_ref, o_ref):
        t = wi_ref[...] * as_ref[...][None, :] + wj_ref[...] * ad_ref[...][None, :]
        s = jnp.sum(t, axis=1, keepdims=True)
        o_ref[...] = jnp.where(s >= 0, s, _NEG_SLOPE * s)

    return pl.pallas_call(
        body,
        grid=(grid,),
        in_specs=[
            pl.BlockSpec((blk, cw), lambda i: (i, 0)),
            pl.BlockSpec((blk, cw), lambda i: (i, 0)),
            pl.BlockSpec((cw,), lambda i: (0,)),
            pl.BlockSpec((cw,), lambda i: (0,)),
        ],
        out_specs=pl.BlockSpec((blk, 1), lambda i: (i, 0)),
        out_shape=jax.ShapeDtypeStruct((n_tc, 1), jnp.float32),
    )


_SC_EDGES_PER_WORKER = 3120


def kernel(Wx_i, Wx_j, a_src, a_dst):
    n, cw = Wx_i.shape
    n_sc = 32 * _SC_EDGES_PER_WORKER
    n_tc = n - n_sc
    sc_k = _make_sc_kernel(n_sc, cw, e0=n_tc)
    tc_k = _make_tc_kernel(n_tc, cw, 4096)
    sc_out = sc_k(Wx_i.reshape(-1), Wx_j.reshape(-1), a_src, a_dst)
    tc_out = tc_k(Wx_i, Wx_j, a_src, a_dst).reshape(-1)
    return jnp.concatenate([tc_out, sc_out])


# speedup vs baseline: 1.6470x; 1.6470x over previous
import functools

import jax
import jax.numpy as jnp
from jax import lax
from jax.experimental import pallas as pl
from jax.experimental.pallas import tpu as pltpu
from jax.experimental.pallas import tpu_sc as plsc

_NEG_SLOPE = 0.2
_L = 16


def _make_sc_kernel(n, cw, e0=0):
    info = plsc.get_sparse_core_info()
    nc, ns = info.num_cores, info.num_subcores
    nw = nc * ns
    ew = n // nw
    ch = 80
    nch = ew // ch
    assert n == nw * ew and ew == ch * nch and cw % _L == 0 and nch % 2 == 1
    nb = 2 * cw // _L

    mesh = plsc.VectorSubcoreMesh(core_axis_name="c", subcore_axis_name="s")
    scratch = (
        [pltpu.VMEM((ch * cw,), jnp.float32)] * 4
        + [
            pltpu.VMEM((2 * cw,), jnp.float32),
            pltpu.VMEM((_L * _L,), jnp.float32),
            pltpu.VMEM((ew,), jnp.float32),
        ]
        + [pltpu.SemaphoreType.DMA] * 4
    )

    @functools.partial(
        pl.kernel,
        mesh=mesh,
        out_type=jax.ShapeDtypeStruct((n,), jnp.float32),
        scratch_types=scratch,
        compiler_params=pltpu.CompilerParams(needs_layout_passes=False),
    )
    def k(wi_hbm, wj_hbm, asrc_hbm, adst_hbm, out_hbm,
          wi0, wi1, wj0, wj1, avec, pbuf, outbuf, si0, si1, sj0, sj1):
        wid = lax.axis_index("s") * nc + lax.axis_index("c")
        base = wid * ew

        pltpu.sync_copy(asrc_hbm, avec.at[pl.ds(0, cw)])
        pltpu.sync_copy(adst_hbm, avec.at[pl.ds(cw, cw)])
        a_regs = [avec[pl.ds(b * _L, _L)] for b in range(nb)]
        tidx = lax.iota(jnp.int32, _L) * _L

        slots = [(wi0, wj0, si0, sj0), (wi1, wj1, si1, sj1)]

        def start(c, slot):
            wi_b, wj_b, si, sj = slots[slot]
            off = (e0 + base + c * ch) * cw
            pltpu.async_copy(wi_hbm.at[pl.ds(off, ch * cw)], wi_b, si)
            pltpu.async_copy(wj_hbm.at[pl.ds(off, ch * cw)], wj_b, sj)

        def wait(c, slot):
            wi_b, wj_b, si, sj = slots[slot]
            off = (e0 + base + c * ch) * cw
            pltpu.make_async_copy(wi_hbm.at[pl.ds(off, ch * cw)], wi_b, si).wait()
            pltpu.make_async_copy(wj_hbm.at[pl.ds(off, ch * cw)], wj_b, sj).wait()

        def compute(c, slot):
            wi_b, wj_b, _, _ = slots[slot]

            def group(g, carry):
                for e in range(_L):
                    ro = (g * _L + e) * cw
                    prods = [wi_b[pl.ds(ro + b * _L, _L)] * a_regs[b]
                             for b in range(nb // 2)]
                    prods += [wj_b[pl.ds(ro + b * _L, _L)] * a_regs[nb // 2 + b]
                              for b in range(nb // 2)]
                    while len(prods) > 1:
                        nxt = [prods[i] + prods[i + 1]
                               for i in range(0, len(prods) - 1, 2)]
                        if len(prods) % 2:
                            nxt.append(prods[-1])
                        prods = nxt
                    pbuf[pl.ds(e * _L, _L)] = prods[0]
                cols = [plsc.load_gather(pbuf, [tidx + l]) for l in range(_L)]
                while len(cols) > 1:
                    cols = [cols[i] + cols[i + 1] for i in range(0, len(cols), 2)]
                r = cols[0]
                r = jnp.where(r >= 0, r, _NEG_SLOPE * r)
                outbuf[pl.ds(c * ch + g * _L, _L)] = r
                return carry

            lax.fori_loop(0, ch // _L, group, 0)

        start(0, 0)

        def body(t, carry):
            c0 = 2 * t
            start(c0 + 1, 1)
            wait(c0, 0)
            compute(c0, 0)
            start(c0 + 2, 0)
            wait(c0 + 1, 1)
            compute(c0 + 1, 1)
            return carry

        lax.fori_loop(0, (nch - 1) // 2, body, 0)
        wait(nch - 1, 0)
        compute(nch - 1, 0)
        pltpu.sync_copy(outbuf, out_hbm.at[pl.ds(base, ew)])

    return k


def _make_tc_kernel(n_tc, cw, blk):
    grid = pl.cdiv(n_tc, blk)

    def body(wi_ref, wj_ref, as_ref, ad_ref, o_ref):
        t = wi_ref[...] * as_ref[...][None, :] + wj_ref[...] * ad_ref[...][None, :]
        s = jnp.sum(t.reshape(blk // cw, cw, cw), axis=2)
        o_ref[...] = jnp.where(s >= 0, s, _NEG_SLOPE * s)

    return pl.pallas_call(
        body,
        grid=(grid,),
        in_specs=[
            pl.BlockSpec((blk, cw), lambda i: (i, 0)),
            pl.BlockSpec((blk, cw), lambda i: (i, 0)),
            pl.BlockSpec((cw,), lambda i: (0,)),
            pl.BlockSpec((cw,), lambda i: (0,)),
        ],
        out_specs=pl.BlockSpec((blk // cw, cw), lambda i: (i, 0)),
        out_shape=jax.ShapeDtypeStruct((n_tc // cw, cw), jnp.float32),
    )


_SC_EDGES_PER_WORKER = 3120


def kernel(Wx_i, Wx_j, a_src, a_dst):
    n, cw = Wx_i.shape
    n_sc = 32 * _SC_EDGES_PER_WORKER
    n_tc = n - n_sc
    sc_k = _make_sc_kernel(n_sc, cw, e0=n_tc)
    tc_k = _make_tc_kernel(n_tc, cw, 4096)
    sc_out = sc_k(Wx_i.reshape(-1), Wx_j.reshape(-1), a_src, a_dst)
    tc_out = tc_k(Wx_i, Wx_j, a_src, a_dst).reshape(-1)
    return jnp.concatenate([tc_out, sc_out])
